# baseline (speedup 1.0000x reference)
import functools

import jax
import jax.numpy as jnp
from jax import lax
from jax.experimental import pallas as pl
from jax.experimental.pallas import tpu as pltpu
from jax.experimental.pallas import tpu_sc as plsc

NC = 2
NS = 16
NW = NC * NS

ROW_BLK = 1024
K = 128

_mesh = functools.partial(
    plsc.VectorSubcoreMesh,
    core_axis_name="c",
    subcore_axis_name="s",
    num_cores=NC,
    num_subcores=NS,
)


def _make_deg_kernel(npad, epad):
  ept = epad // NW
  nchunk = ept // K
  rows_per_s = npad // NS

  @functools.partial(
      pl.kernel,
      out_type=jax.ShapeDtypeStruct((NC, npad), jnp.float32),
      mesh=_mesh(),
      scratch_types=[
          pltpu.VMEM_SHARED((npad,), jnp.float32),
          pltpu.VMEM((K,), jnp.int32),
          pltpu.VMEM((K,), jnp.float32),
          pltpu.VMEM((rows_per_s,), jnp.float32),
      ],
  )
  def deg_kernel(col_hbm, ew_hbm, degp_hbm, deg_sh, cidx_buf, ew_buf, zeros_v):
    cid = lax.axis_index("c")
    sid = lax.axis_index("s")
    wid = sid * NC + cid

    def zfill(i, _):
      zeros_v[pl.ds(i * 16, 16)] = jnp.zeros((16,), jnp.float32)
      return 0
    lax.fori_loop(0, rows_per_s // 16, zfill, 0)
    pltpu.sync_copy(zeros_v, deg_sh.at[pl.ds(sid * rows_per_s, rows_per_s)])
    plsc.subcore_barrier()

    def chunk(t, _):
      base = wid * ept + t * K
      pltpu.sync_copy(col_hbm.at[pl.ds(base, K)], cidx_buf)
      pltpu.sync_copy(ew_hbm.at[pl.ds(base, K)], ew_buf)
      pltpu.sync_copy(ew_buf, deg_sh.at[cidx_buf], add=True)
      return 0
    lax.fori_loop(0, nchunk, chunk, 0)
    plsc.subcore_barrier()

    pltpu.sync_copy(
        deg_sh.at[pl.ds(sid * rows_per_s, rows_per_s)],
        degp_hbm.at[cid, pl.ds(sid * rows_per_s, rows_per_s)],
    )

  return deg_kernel


def _make_edge_kernel(npad, d, epad):
  ept = epad // NW
  nchunk = ept // K
  rows_per_s = npad // NS

  @functools.partial(
      pl.kernel,
      out_type=jax.ShapeDtypeStruct((NC, npad, d), jnp.float32),
      mesh=_mesh(),
      scratch_types=[
          pltpu.VMEM_SHARED((npad, d), jnp.float32),
          pltpu.VMEM((2 * K,), jnp.int32),
          pltpu.VMEM((K,), jnp.int32),
          pltpu.VMEM((K, d), jnp.float32),
          pltpu.SemaphoreType.DMA,
      ],
  )
  def edge_kernel(g_hbm, rw_hbm, col_hbm, sp_hbm,
                  acc_sh, rw_buf, cidx_v, rows_v, sem):
    cid = lax.axis_index("c")
    sid = lax.axis_index("s")
    wid = sid * NC + cid
    nsub = d // 16

    def zfill(i, _):
      def zrow(cc, _):
        rows_v[i, pl.ds(cc * 16, 16)] = jnp.zeros((16,), jnp.float32)
        return 0
      lax.fori_loop(0, nsub, zrow, 0)
      return 0
    lax.fori_loop(0, K, zfill, 0)
    def zcopy(i, _):
      pltpu.sync_copy(rows_v, acc_sh.at[pl.ds(sid * rows_per_s + i * K, K)])
      return 0
    lax.fori_loop(0, rows_per_s // K, zcopy, 0)
    plsc.subcore_barrier()

    def chunk(t, _):
      pltpu.sync_copy(rw_hbm.at[pl.ds(2 * (wid * ept + t * K), 2 * K)], rw_buf)
      pltpu.sync_copy(col_hbm.at[pl.ds(wid * ept + t * K, K)], cidx_v)
      pltpu.async_copy(g_hbm.at[rw_buf.at[pl.ds(0, K)]], rows_v, sem).wait()
      def scale16(jj, _):
        ew16 = lax.bitcast_convert_type(rw_buf[pl.ds(K + jj * 16, 16)], jnp.float32)
        base_j = jj * 16
        for lane in range(16):
          w = ew16[lane]
          j = base_j + lane
          for cc in range(nsub):
            sl = pl.ds(cc * 16, 16)
            rows_v[j, sl] = rows_v[j, sl] * w
        return 0
      lax.fori_loop(0, K // 16, scale16, 0)
      pltpu.sync_copy(rows_v, acc_sh.at[cidx_v], add=True)
      return 0
    lax.fori_loop(0, nchunk, chunk, 0)
    plsc.subcore_barrier()

    pltpu.sync_copy(
        acc_sh.at[pl.ds(sid * rows_per_s, rows_per_s)],
        sp_hbm.at[cid, pl.ds(sid * rows_per_s, rows_per_s)],
    )

  return edge_kernel


def _dis_from(degp_blk):
  deg = 1.0 + degp_blk[0, :] + degp_blk[1, :]
  return lax.rsqrt(deg)


def _mm_body(x_ref, w_ref, h_ref):
  h_ref[...] = jnp.dot(x_ref[...], w_ref[...],
                       preferred_element_type=jnp.float32)


def _g1_body(h_ref, degp_ref, g_ref):
  dis = _dis_from(degp_ref[...])
  g_ref[...] = h_ref[...] * dis[:, None]


def _g2_body(sp_ref, g1_ref, degp_ref, b1_ref, w2_ref, g2_ref):
  dis = _dis_from(degp_ref[...])
  pre = (sp_ref[0] + sp_ref[1] + g1_ref[...]) * dis[:, None] + b1_ref[...]
  h = jnp.maximum(pre, 0.0)
  hw = jnp.dot(h, w2_ref[...], preferred_element_type=jnp.float32)
  g2_ref[...] = hw * dis[:, None]


def _out_body(sp_ref, g2_ref, degp_ref, b2_ref, out_ref):
  dis = _dis_from(degp_ref[...])
  out_ref[...] = (sp_ref[0] + sp_ref[1] + g2_ref[...]) * dis[:, None] + b2_ref[...]


def kernel(x, edge_index, edge_weight, W1, b1, W2, b2):
  n, d = x.shape
  e = edge_index.shape[1]
  npad = ((n + ROW_BLK - 1) // ROW_BLK) * ROW_BLK
  assert npad % (NS * K) == 0
  epad = ((e + NW * K - 1) // (NW * K)) * (NW * K)

  row = edge_index[0].astype(jnp.int32)
  col = edge_index[1].astype(jnp.int32)
  ew = edge_weight.astype(jnp.float32)
  pe = epad - e
  if pe:
    pad_idx = jnp.full((pe,), npad - 1, jnp.int32)
    row = jnp.concatenate([row, pad_idx])
    col = jnp.concatenate([col, pad_idx])
    ew = jnp.concatenate([ew, jnp.zeros((pe,), jnp.float32)])
  ew_bits = lax.bitcast_convert_type(ew, jnp.int32)
  rw = jnp.concatenate(
      [row.reshape(-1, K), ew_bits.reshape(-1, K)], axis=1).reshape(-1)
  x_p = jnp.pad(x, ((0, npad - n), (0, 0)))
  b1_2d = b1.reshape(1, d)
  b2_2d = b2.reshape(1, d)

  deg_kernel = _make_deg_kernel(npad, epad)
  edge_kernel = _make_edge_kernel(npad, d, epad)

  grid = npad // ROW_BLK
  blk_rows = pl.BlockSpec((ROW_BLK, d), lambda i: (i, 0))
  blk_deg = pl.BlockSpec((NC, ROW_BLK), lambda i: (0, i))
  blk_sp = pl.BlockSpec((NC, ROW_BLK, d), lambda i: (0, i, 0))
  blk_w = pl.BlockSpec((d, d), lambda i: (0, 0))
  blk_b = pl.BlockSpec((1, d), lambda i: (0, 0))

  degp = deg_kernel(col, ew)
  h1 = pl.pallas_call(
      _mm_body,
      grid=(grid,),
      in_specs=[blk_rows, blk_w],
      out_specs=blk_rows,
      out_shape=jax.ShapeDtypeStruct((npad, d), jnp.float32),
  )(x_p, W1)
  g1 = pl.pallas_call(
      _g1_body,
      grid=(grid,),
      in_specs=[blk_rows, blk_deg],
      out_specs=blk_rows,
      out_shape=jax.ShapeDtypeStruct((npad, d), jnp.float32),
  )(h1, degp)

  s1 = edge_kernel(g1, rw, col)

  g2 = pl.pallas_call(
      _g2_body,
      grid=(grid,),
      in_specs=[blk_sp, blk_rows, blk_deg, blk_b, blk_w],
      out_specs=blk_rows,
      out_shape=jax.ShapeDtypeStruct((npad, d), jnp.float32),
  )(s1, g1, degp, b1_2d, W2)

  s2 = edge_kernel(g2, rw, col)

  out = pl.pallas_call(
      _out_body,
      grid=(grid,),
      in_specs=[blk_sp, blk_rows, blk_deg, blk_b],
      out_specs=blk_rows,
      out_shape=jax.ShapeDtypeStruct((npad, d), jnp.float32),
  )(s2, g2, degp, b2_2d)

  return out[:n]


# speedup vs baseline: 1.0426x; 1.0426x over previous
import functools

import jax
import jax.numpy as jnp
from jax import lax
from jax.experimental import pallas as pl
from jax.experimental.pallas import tpu as pltpu
from jax.experimental.pallas import tpu_sc as plsc

NC = 2
NS = 16
NW = NC * NS

ROW_BLK = 1024
K = 128

_mesh = functools.partial(
    plsc.VectorSubcoreMesh,
    core_axis_name="c",
    subcore_axis_name="s",
    num_cores=NC,
    num_subcores=NS,
)


def _make_deg_kernel(npad, epad):
  ept = epad // NW
  nchunk = ept // K
  rows_per_s = npad // NS

  @functools.partial(
      pl.kernel,
      out_type=jax.ShapeDtypeStruct((NC, npad), jnp.float32),
      mesh=_mesh(),
      scratch_types=[
          pltpu.VMEM_SHARED((npad,), jnp.float32),
          pltpu.VMEM((K,), jnp.int32),
          pltpu.VMEM((K,), jnp.float32),
          pltpu.VMEM((rows_per_s,), jnp.float32),
      ],
  )
  def deg_kernel(col_hbm, ew_hbm, degp_hbm, deg_sh, cidx_buf, ew_buf, zeros_v):
    cid = lax.axis_index("c")
    sid = lax.axis_index("s")
    wid = sid * NC + cid

    def zfill(i, _):
      zeros_v[pl.ds(i * 16, 16)] = jnp.zeros((16,), jnp.float32)
      return 0
    lax.fori_loop(0, rows_per_s // 16, zfill, 0)
    pltpu.sync_copy(zeros_v, deg_sh.at[pl.ds(sid * rows_per_s, rows_per_s)])
    plsc.subcore_barrier()

    def chunk(t, _):
      base = wid * ept + t * K
      pltpu.sync_copy(col_hbm.at[pl.ds(base, K)], cidx_buf)
      pltpu.sync_copy(ew_hbm.at[pl.ds(base, K)], ew_buf)
      pltpu.sync_copy(ew_buf, deg_sh.at[cidx_buf], add=True)
      return 0
    lax.fori_loop(0, nchunk, chunk, 0)
    plsc.subcore_barrier()

    pltpu.sync_copy(
        deg_sh.at[pl.ds(sid * rows_per_s, rows_per_s)],
        degp_hbm.at[cid, pl.ds(sid * rows_per_s, rows_per_s)],
    )

  return deg_kernel


def _make_edge_kernel(npad, d, epad):
  ept = epad // NW
  nchunk = ept // K
  rows_per_s = npad // NS

  @functools.partial(
      pl.kernel,
      out_type=jax.ShapeDtypeStruct((NC, npad, d), jnp.float32),
      mesh=_mesh(),
      scratch_types=[
          pltpu.VMEM_SHARED((npad, d), jnp.float32),
          pltpu.VMEM((2 * K,), jnp.int32),
          pltpu.VMEM((K,), jnp.int32),
          pltpu.VMEM((K, d), jnp.float32),
          pltpu.SemaphoreType.DMA,
      ],
  )
  def edge_kernel(g_hbm, rw_hbm, col_hbm, sp_hbm,
                  acc_sh, rw_buf, cidx_v, rows_v, sem):
    cid = lax.axis_index("c")
    sid = lax.axis_index("s")
    wid = sid * NC + cid
    nsub = d // 16

    def zfill(i, _):
      def zrow(cc, _):
        rows_v[i, pl.ds(cc * 16, 16)] = jnp.zeros((16,), jnp.float32)
        return 0
      lax.fori_loop(0, nsub, zrow, 0)
      return 0
    lax.fori_loop(0, K, zfill, 0)
    def zcopy(i, _):
      pltpu.sync_copy(rows_v, acc_sh.at[pl.ds(sid * rows_per_s + i * K, K)])
      return 0
    lax.fori_loop(0, rows_per_s // K, zcopy, 0)
    plsc.subcore_barrier()

    def chunk(t, _):
      pltpu.sync_copy(rw_hbm.at[pl.ds(2 * (wid * ept + t * K), 2 * K)], rw_buf)
      pltpu.sync_copy(col_hbm.at[pl.ds(wid * ept + t * K, K)], cidx_v)
      pltpu.async_copy(g_hbm.at[rw_buf.at[pl.ds(0, K)]], rows_v, sem).wait()
      def scale16(jj, _):
        ew16 = lax.bitcast_convert_type(rw_buf[pl.ds(K + jj * 16, 16)], jnp.float32)
        base_j = jj * 16
        for lane in range(16):
          w = ew16[lane]
          j = base_j + lane
          for cc in range(nsub):
            sl = pl.ds(cc * 16, 16)
            rows_v[j, sl] = rows_v[j, sl] * w
        return 0
      lax.fori_loop(0, K // 16, scale16, 0)
      pltpu.sync_copy(rows_v, acc_sh.at[cidx_v], add=True)
      return 0
    lax.fori_loop(0, nchunk, chunk, 0)
    plsc.subcore_barrier()

    pltpu.sync_copy(
        acc_sh.at[pl.ds(sid * rows_per_s, rows_per_s)],
        sp_hbm.at[cid, pl.ds(sid * rows_per_s, rows_per_s)],
    )

  return edge_kernel


def _dis_from(degp_blk):
  deg = 1.0 + degp_blk[0, :] + degp_blk[1, :]
  return lax.rsqrt(deg)


def _g1_body(x_ref, w_ref, degp_ref, g_ref):
  h = jnp.dot(x_ref[...], w_ref[...], preferred_element_type=jnp.float32)
  dis = _dis_from(degp_ref[...])
  g_ref[...] = h * dis[:, None]


def _g2_body(sp_ref, g1_ref, degp_ref, b1_ref, w2_ref, g2_ref):
  dis = _dis_from(degp_ref[...])
  pre = (sp_ref[0] + sp_ref[1] + g1_ref[...]) * dis[:, None] + b1_ref[...]
  h = jnp.maximum(pre, 0.0)
  hw = jnp.dot(h, w2_ref[...], preferred_element_type=jnp.float32)
  g2_ref[...] = hw * dis[:, None]


def _out_body(sp_ref, g2_ref, degp_ref, b2_ref, out_ref):
  dis = _dis_from(degp_ref[...])
  out_ref[...] = (sp_ref[0] + sp_ref[1] + g2_ref[...]) * dis[:, None] + b2_ref[...]


def kernel(x, edge_index, edge_weight, W1, b1, W2, b2):
  n, d = x.shape
  e = edge_index.shape[1]
  npad = ((n + ROW_BLK - 1) // ROW_BLK) * ROW_BLK
  assert npad % (NS * K) == 0
  epad = ((e + NW * K - 1) // (NW * K)) * (NW * K)

  row = edge_index[0].astype(jnp.int32)
  col = edge_index[1].astype(jnp.int32)
  ew = edge_weight.astype(jnp.float32)
  pe = epad - e
  if pe:
    pad_idx = jnp.full((pe,), npad - 1, jnp.int32)
    row = jnp.concatenate([row, pad_idx])
    col = jnp.concatenate([col, pad_idx])
    ew = jnp.concatenate([ew, jnp.zeros((pe,), jnp.float32)])
  ew_bits = lax.bitcast_convert_type(ew, jnp.int32)
  rw = jnp.concatenate(
      [row.reshape(-1, K), ew_bits.reshape(-1, K)], axis=1).reshape(-1)
  x_p = jnp.pad(x, ((0, npad - n), (0, 0)))
  b1_2d = b1.reshape(1, d)
  b2_2d = b2.reshape(1, d)

  deg_kernel = _make_deg_kernel(npad, epad)
  edge_kernel = _make_edge_kernel(npad, d, epad)

  grid = npad // ROW_BLK
  blk_rows = pl.BlockSpec((ROW_BLK, d), lambda i: (i, 0))
  blk_deg = pl.BlockSpec((NC, ROW_BLK), lambda i: (0, i))
  blk_sp = pl.BlockSpec((NC, ROW_BLK, d), lambda i: (0, i, 0))
  blk_w = pl.BlockSpec((d, d), lambda i: (0, 0))
  blk_b = pl.BlockSpec((1, d), lambda i: (0, 0))

  degp = deg_kernel(col, ew)
  g1 = pl.pallas_call(
      _g1_body,
      grid=(grid,),
      in_specs=[blk_rows, blk_w, blk_deg],
      out_specs=blk_rows,
      out_shape=jax.ShapeDtypeStruct((npad, d), jnp.float32),
  )(x_p, W1, degp)

  s1 = edge_kernel(g1, rw, col)

  g2 = pl.pallas_call(
      _g2_body,
      grid=(grid,),
      in_specs=[blk_sp, blk_rows, blk_deg, blk_b, blk_w],
      out_specs=blk_rows,
      out_shape=jax.ShapeDtypeStruct((npad, d), jnp.float32),
  )(s1, g1, degp, b1_2d, W2)

  s2 = edge_kernel(g2, rw, col)

  out = pl.pallas_call(
      _out_body,
      grid=(grid,),
      in_specs=[blk_sp, blk_rows, blk_deg, blk_b],
      out_specs=blk_rows,
      out_shape=jax.ShapeDtypeStruct((npad, d), jnp.float32),
  )(s2, g2, degp, b2_2d)

  return out[:n]
